# Initial kernel scaffold; baseline (speedup 1.0000x reference)
import jax
import jax.numpy as jnp
from jax.experimental import pallas as pl


def kernel(x, edge_index, batch, params):
    raise NotImplementedError("write your pallas kernel here")


# speedup vs baseline: 4.3545x; 4.3545x over previous
import functools

import jax
import jax.numpy as jnp
from jax import lax
from jax.experimental import pallas as pl
from jax.experimental.pallas import tpu as pltpu
from jax.experimental.pallas import tpu_sc as plsc

NN = 10000
EE = 320000
DIN = 128
DD = 512
GG = 64
OUTD = 16

N2 = 10240
RB = 1024
NBLK = N2 // RB
CH = 128
NCH = DD // CH
BE = 125
NROW = EE // BE
NC, NS = 2, 16
RPS = N2 // NS
WB = 128
NWB = RPS // WB


@functools.lru_cache(maxsize=None)
def _make_sc_agg(n_chunks):
    n_out = 4 if n_chunks == 4 else 2
    iters = NROW // NS if n_chunks == 4 else NROW // (NC * NS)
    n_tasks = 2 if n_chunks == 4 else 1
    mesh = plsc.VectorSubcoreMesh(core_axis_name="c", subcore_axis_name="s",
                                  num_cores=NC, num_subcores=NS)

    @functools.partial(
        pl.kernel,
        out_type=jax.ShapeDtypeStruct((n_out, N2, CH), jnp.float32),
        mesh=mesh,
        scratch_types=[
            pltpu.VMEM((8, BE), jnp.int32),
            pltpu.VMEM((8, BE), jnp.int32),
            pltpu.VMEM((BE, CH), jnp.float32),
            pltpu.VMEM((WB, CH), jnp.float32),
            pltpu.VMEM_SHARED((N2, CH), jnp.float32),
            pltpu.SemaphoreType.DMA,
        ],
    )
    def k(h_hbm, src_hbm, dst_hbm, zeros_hbm, out_hbm,
          sidx, didx, rows, stage, acc, sem):
        c = lax.axis_index("c")
        s = lax.axis_index("s")
        if n_chunks == 4:
            rowbase = s * iters
        else:
            rowbase = (c * NS + s) * iters
        for t in range(n_tasks):
            if n_chunks == 4:
                chunk = c + 2 * t
                oi = chunk
            else:
                chunk = 0
                oi = c
            pltpu.sync_copy(zeros_hbm, stage)
            for w in range(NWB):
                pltpu.sync_copy(stage, acc.at[pl.ds(s * RPS + w * WB, WB)])
            plsc.subcore_barrier()
            table = h_hbm.at[chunk]

            def body(sb, carry):
                base = rowbase + sb * 8
                pltpu.sync_copy(src_hbm.at[pl.ds(base, 8)], sidx)
                pltpu.sync_copy(dst_hbm.at[pl.ds(base, 8)], didx)
                for j in range(8):
                    pltpu.async_copy(table.at[sidx.at[j]], rows, sem).wait()
                    pltpu.sync_copy(rows, acc.at[didx.at[j]], add=True)
                return carry

            lax.fori_loop(0, iters // 8, body, 0)
            plsc.subcore_barrier()
            for w in range(NWB):
                pltpu.sync_copy(acc.at[pl.ds(s * RPS + w * WB, WB)], stage)
                pltpu.sync_copy(stage, out_hbm.at[oi].at[pl.ds(s * RPS + w * WB, WB)])
            plsc.subcore_barrier()

    return k


def _make_mlp(nch, nagg, din, prec=None):
    def body(hr, ar, w1r, b1r, w2r, b2r, z3r, statsr):
        i = pl.program_id(0)
        if nch == 1:
            z = hr[0] + ar[0] + ar[1]
        else:
            z = jnp.concatenate([hr[cc] + ar[cc] for cc in range(nch)], axis=-1)
        z1 = jnp.maximum(
            jnp.dot(z, w1r[...], preferred_element_type=jnp.float32,
                    precision=prec) + b1r[...], 0.0)
        z2 = jnp.dot(z1, w2r[...], preferred_element_type=jnp.float32,
                     precision=prec) + b2r[...]
        z3 = jnp.maximum(z2, 0.0)
        z3r[...] = z3
        rowid = lax.broadcasted_iota(jnp.int32, (RB, DD), 0) + i * RB
        z3m = jnp.where(rowid < NN, z3, 0.0)
        s1 = jnp.sum(z3m, axis=0, keepdims=True)

        @pl.when(i == 0)
        def _():
            statsr[...] = jnp.zeros_like(statsr)

        statsr[0:1, :] += s1

    return pl.pallas_call(
        body,
        grid=(NBLK,),
        in_specs=[
            pl.BlockSpec((nch, RB, CH), lambda i: (0, i, 0)),
            pl.BlockSpec((nagg, RB, CH), lambda i: (0, i, 0)),
            pl.BlockSpec((din, DD), lambda i: (0, 0)),
            pl.BlockSpec((1, DD), lambda i: (0, 0)),
            pl.BlockSpec((DD, DD), lambda i: (0, 0)),
            pl.BlockSpec((1, DD), lambda i: (0, 0)),
        ],
        out_specs=[
            pl.BlockSpec((RB, DD), lambda i: (i, 0)),
            pl.BlockSpec((8, DD), lambda i: (0, 0)),
        ],
        out_shape=[
            jax.ShapeDtypeStruct((N2, DD), jnp.float32),
            jax.ShapeDtypeStruct((8, DD), jnp.float32),
        ],
    )


def _make_var():
    def body(z3r, statsr, vr):
        i = pl.program_id(0)
        mean = statsr[0:1, :] * (1.0 / NN)
        dv = z3r[...] - mean
        rowid = lax.broadcasted_iota(jnp.int32, (RB, DD), 0) + i * RB
        dv = jnp.where(rowid < NN, dv, 0.0)
        s = jnp.sum(dv * dv, axis=0, keepdims=True)

        @pl.when(i == 0)
        def _():
            vr[...] = jnp.zeros_like(vr)

        vr[0:1, :] += s

    return pl.pallas_call(
        body,
        grid=(NBLK,),
        in_specs=[
            pl.BlockSpec((RB, DD), lambda i: (i, 0)),
            pl.BlockSpec((8, DD), lambda i: (0, 0)),
        ],
        out_specs=pl.BlockSpec((8, DD), lambda i: (0, 0)),
        out_shape=jax.ShapeDtypeStruct((8, DD), jnp.float32),
    )


def _norm_common(z3r, statsr, varr, gr, br):
    z = z3r[...]
    mean = statsr[0:1, :] * (1.0 / NN)
    var = varr[0:1, :] * (1.0 / NN)
    return (z - mean) / jnp.sqrt(var + 1e-5) * gr[...] + br[...]


def _make_norm_rechunk():
    def body(z3r, statsr, varr, gr, br, outr):
        h = _norm_common(z3r, statsr, varr, gr, br)
        for cc in range(NCH):
            outr[cc] = h[:, cc * CH:(cc + 1) * CH]

    return pl.pallas_call(
        body,
        grid=(NBLK,),
        in_specs=[
            pl.BlockSpec((RB, DD), lambda i: (i, 0)),
            pl.BlockSpec((8, DD), lambda i: (0, 0)),
            pl.BlockSpec((8, DD), lambda i: (0, 0)),
            pl.BlockSpec((1, DD), lambda i: (0, 0)),
            pl.BlockSpec((1, DD), lambda i: (0, 0)),
        ],
        out_specs=pl.BlockSpec((NCH, RB, CH), lambda i: (0, i, 0)),
        out_shape=jax.ShapeDtypeStruct((NCH, N2, CH), jnp.float32),
    )


def _make_norm_pool():
    def body(z3r, statsr, varr, gr, br, batchr, sumsr, cntr):
        i = pl.program_id(0)
        h = _norm_common(z3r, statsr, varr, gr, br)
        onehot = (batchr[...] ==
                  lax.broadcasted_iota(jnp.int32, (GG, RB), 0)).astype(jnp.float32)
        ps = jnp.dot(onehot, h, preferred_element_type=jnp.float32,
                     precision=lax.Precision.HIGHEST)
        cnt = jnp.sum(onehot, axis=1, keepdims=True)

        @pl.when(i == 0)
        def _():
            sumsr[...] = jnp.zeros_like(sumsr)
            cntr[...] = jnp.zeros_like(cntr)

        sumsr[...] += ps
        cntr[...] += jnp.broadcast_to(cnt, (GG, CH))

    return pl.pallas_call(
        body,
        grid=(NBLK,),
        in_specs=[
            pl.BlockSpec((RB, DD), lambda i: (i, 0)),
            pl.BlockSpec((8, DD), lambda i: (0, 0)),
            pl.BlockSpec((8, DD), lambda i: (0, 0)),
            pl.BlockSpec((1, DD), lambda i: (0, 0)),
            pl.BlockSpec((1, DD), lambda i: (0, 0)),
            pl.BlockSpec((1, RB), lambda i: (0, i)),
        ],
        out_specs=[
            pl.BlockSpec((GG, DD), lambda i: (0, 0)),
            pl.BlockSpec((GG, CH), lambda i: (0, 0)),
        ],
        out_shape=[
            jax.ShapeDtypeStruct((GG, DD), jnp.float32),
            jax.ShapeDtypeStruct((GG, CH), jnp.float32),
        ],
    )


def _make_head():
    def body(sumsr, cntr, w1r, b1r, w2r, b2r, outr):
        cnt = jnp.maximum(cntr[...][:, 0:1], 1.0)
        pooled = sumsr[...] / cnt
        feat = jnp.tanh(
            jnp.dot(pooled, w1r[...], preferred_element_type=jnp.float32,
                    precision=lax.Precision.HIGHEST) + b1r[...])
        outr[...] = (
            jnp.dot(feat, w2r[...], preferred_element_type=jnp.float32,
                    precision=lax.Precision.HIGHEST) + b2r[...])

    return pl.pallas_call(
        body,
        out_shape=jax.ShapeDtypeStruct((GG, CH), jnp.float32),
    )


_MLP0 = _make_mlp(1, 2, DIN)
_MLP = _make_mlp(NCH, NCH, DD)
_VAR = _make_var()
_NORM_RECHUNK = _make_norm_rechunk()
_NORM_POOL = _make_norm_pool()
_HEAD = _make_head()


def kernel(x, edge_index, batch, params):
    f32 = jnp.float32
    src2 = edge_index[0].reshape(NROW, BE)
    dst2 = edge_index[1].reshape(NROW, BE)
    xp = jnp.zeros((1, N2, CH), f32).at[0, :NN].set(x)
    batch_pad = jnp.concatenate(
        [batch, jnp.full((N2 - NN,), GG, jnp.int32)]).reshape(1, N2)
    zeros_tile = jnp.zeros((WB, CH), f32)

    logw = jnp.zeros((DD, CH), f32).at[:, :OUTD].set(params['logits_W'])
    logb = jnp.zeros((1, CH), f32).at[0, :OUTD].set(params['logits_b'])

    h = xp
    z3 = stats = None
    for i in range(5):
        agg = _make_sc_agg(1 if i == 0 else 4)(h, src2, dst2, zeros_tile)
        w1 = params[f'conv{i}_W1']
        b1 = params[f'conv{i}_b1'].reshape(1, DD)
        w2 = params[f'conv{i}_W2']
        b2 = params[f'conv{i}_b2'].reshape(1, DD)
        z3, stats = (_MLP0 if i == 0 else _MLP)(h, agg, w1, b1, w2, b2)
        var = _VAR(z3, stats)
        g = params[f'bn{i}_g'].reshape(1, DD)
        bb = params[f'bn{i}_b'].reshape(1, DD)
        if i < 4:
            h = _NORM_RECHUNK(z3, stats, var, g, bb)
        else:
            sums, cnts = _NORM_POOL(z3, stats, var, g, bb, batch_pad)
    out = _HEAD(sums, cnts, params['fc1_W'], params['fc1_b'].reshape(1, DD),
                logw, logb)
    return out[:, :OUTD]
